# baseline (speedup 1.0000x reference)
import functools
import math

import jax
import jax.numpy as jnp
import numpy as np
from jax import lax
from jax.experimental import pallas as pl
from jax.experimental.pallas import tpu as pltpu
from jax.experimental.pallas import tpu_sc as plsc

_phi = (1.0 + np.sqrt(5.0)) / 2.0
_verts = np.array([[-1.0, _phi, 0.0], [1.0, _phi, 0.0], [-1.0, -_phi, 0.0],
                   [1.0, -_phi, 0.0], [0.0, -1.0, _phi], [0.0, 1.0, _phi],
                   [0.0, -1.0, -_phi], [0.0, 1.0, -_phi], [_phi, 0.0, -1.0],
                   [_phi, 0.0, 1.0], [-_phi, 0.0, -1.0], [-_phi, 0.0, 1.0]],
                  dtype=np.float32)
_VS = _verts / np.linalg.norm(_verts, axis=-1, keepdims=True)
_ROLL = np.array([[0, 1, 2, 3, 4, 5], [0, 2, 3, 4, 5, 1], [0, 3, 4, 5, 1, 2],
                  [0, 4, 5, 1, 2, 3], [0, 5, 1, 2, 3, 4]])
_CC = np.array([[0, 1, 2, 3, 4, 5], [0, 5, 4, 3, 2, 1], [2, 5, 4, 1, 0, 3],
                [5, 0, 1, 3, 2, 4], [3, 5, 2, 0, 4, 1], [4, 0, 5, 2, 1, 3],
                [1, 3, 4, 2, 0, 5], [1, 0, 2, 4, 3, 5], [4, 1, 2, 5, 0, 3],
                [3, 1, 4, 0, 2, 5], [5, 1, 0, 4, 2, 3], [2, 4, 5, 3, 0, 1]])
_C2V = np.array([[0, 1], [6, 7], [2, 11], [4, 9], [5, 8], [3, 10]])
_INV = np.argsort(_CC, axis=-1)

V = 10000
N = 16
P_PAD = 10240
BP = 512
NBLK = P_PAD // BP
B_TOT = V * N
NW = 32
B_PAD = 163840
BPW = B_PAD // NW
VPAD = 10240


def _sc_gather_build():
    mesh = plsc.VectorSubcoreMesh(core_axis_name="c", subcore_axis_name="s")

    @functools.partial(
        pl.kernel,
        mesh=mesh,
        compiler_params=pltpu.CompilerParams(needs_layout_passes=False),
        out_type=jax.ShapeDtypeStruct((3 * B_PAD,), jnp.float32),
        scratch_types=[
            pltpu.VMEM((3 * VPAD,), jnp.float32),
            pltpu.VMEM((BPW,), jnp.int32),
            pltpu.VMEM((3 * BPW,), jnp.float32),
        ],
    )
    def sc_gather(table_hbm, idx_hbm, out_hbm, tab_v, idx_v, rows_v):
        wid = lax.axis_index("s") * 2 + lax.axis_index("c")
        pltpu.sync_copy(table_hbm, tab_v)
        pltpu.sync_copy(idx_hbm.at[pl.ds(wid * BPW, BPW)], idx_v)

        def body(j, carry):
            o = pl.multiple_of(j * 16, 16)
            iv = idx_v[pl.ds(o, 16)]
            for c in range(3):
                g = plsc.load_gather(tab_v, [iv + c * VPAD])
                rows_v[pl.ds(pl.multiple_of(c * BPW + o, 16), 16)] = g
            return carry

        lax.fori_loop(0, BPW // 16, body, 0)
        for c in range(3):
            pltpu.sync_copy(
                rows_v.at[pl.ds(c * BPW, BPW)],
                out_hbm.at[pl.ds(c * B_PAD + wid * BPW, BPW)],
            )

    return sc_gather


_sc_cache = []


def _get_sc_gather():
    if not _sc_cache:
        _sc_cache.append(_sc_gather_build())
    return _sc_cache[0]


def _tc_body(nbr_ref, ctr_ref, a_ref, a2_ref, j0_ref, b2_ref, out_ref,
             nf_ref, de_ref):
    nbr = nbr_ref[...]
    ctr = ctr_ref[...]
    nd0 = nbr[0] - ctr[0][None, :]
    nd1 = nbr[1] - ctr[1][None, :]
    nd2 = nbr[2] - ctr[2][None, :]
    n2 = nd0 * nd0 + nd1 * nd1 + nd2 * nd2
    rn = lax.rsqrt(jnp.maximum(n2, 1e-24))
    dv = []
    for v in range(12):
        c0, c1, c2 = float(_VS[v, 0]), float(_VS[v, 1]), float(_VS[v, 2])
        dv.append((c0 * nd0 + c1 * nd1 + c2 * nd2) * rn)
    de6 = [jnp.maximum(dv[int(_C2V[c, 0])], dv[int(_C2V[c, 1])])
           for c in range(6)]
    s_all = ((de6[0] + de6[1]) + (de6[2] + de6[3])) + (de6[4] + de6[5])
    for c in range(6):
        de_ref[c] = de6[c]
    de_ref[6] = s_all

    nf_ref[:, 78:80, :] = jnp.zeros((12, 2, BP), jnp.float32)

    def piv_body(r, carry):
        j0 = j0_ref[r]
        dj = de_ref[j0]
        s = de_ref[6]
        for t, k in ((0, 0), (1, 1), (2, 12)):
            act = jnp.maximum(a2_ref[t, 0] * dj + a2_ref[t, 1] * s, 0.0)
            for i in range(6):
                nf_ref[r, k * 6 + i, :] = jnp.sum(act * de6[i], axis=0)
        return carry

    lax.fori_loop(0, 12, piv_body, 0)

    def sym_body(rm, carry):
        r = rm // 5
        k = (rm - r * 5) * 2 + 2
        act1 = (a_ref[r, k, 0] * de6[0] + a_ref[r, k, 1] * de6[1] +
                a_ref[r, k, 2] * de6[2] + a_ref[r, k, 3] * de6[3] +
                a_ref[r, k, 4] * de6[4] + a_ref[r, k, 5] * de6[5])
        act1 = jnp.maximum(act1, 0.0)
        k2 = k + 1
        act2 = (a_ref[r, k2, 0] * de6[0] + a_ref[r, k2, 1] * de6[1] +
                a_ref[r, k2, 2] * de6[2] + a_ref[r, k2, 3] * de6[3] +
                a_ref[r, k2, 4] * de6[4] + a_ref[r, k2, 5] * de6[5])
        act2 = jnp.maximum(act2, 0.0)
        for i in range(6):
            d = de6[i]
            nf_ref[r, k * 6 + i, :] = jnp.sum(act1 * d, axis=0)
            nf_ref[r, k2 * 6 + i, :] = jnp.sum(act2 * d, axis=0)
        return carry

    lax.fori_loop(0, 60, sym_body, 0)

    dims = (((0,), (0,)), ((), ()))
    for c in range(6):
        r0, r1 = int(_C2V[c, 0]), int(_C2V[c, 1])
        foa = lax.dot_general(b2_ref[r0], nf_ref[r0], dims,
                              preferred_element_type=jnp.float32)
        fob = lax.dot_general(b2_ref[r1], nf_ref[r1], dims,
                              preferred_element_type=jnp.float32)
        out_ref[c] = jnp.maximum(foa, fob)


def _tc_build(interpret=False):
    return pl.pallas_call(
        _tc_body,
        grid=(NBLK,),
        in_specs=[
            pl.BlockSpec((3, 16, BP), lambda i: (0, 0, i)),
            pl.BlockSpec((3, BP), lambda i: (0, i)),
            pl.BlockSpec(memory_space=pltpu.SMEM),
            pl.BlockSpec(memory_space=pltpu.SMEM),
            pl.BlockSpec(memory_space=pltpu.SMEM),
            pl.BlockSpec((12, 80, 128), lambda i: (0, 0, 0)),
        ],
        out_specs=pl.BlockSpec((6, 128, BP), lambda i: (0, 0, i)),
        out_shape=jax.ShapeDtypeStruct((6, 128, P_PAD), jnp.float32),
        scratch_shapes=[
            pltpu.VMEM((12, 80, BP), jnp.float32),
            pltpu.VMEM((7, 16, BP), jnp.float32),
        ],
        interpret=interpret,
    )


_tc_call = _tc_build()


def _rows13(w18):
    top = jnp.stack([w18[..., 0]] + [w18[..., 1]] * 5, axis=-1)
    bot = jnp.stack([w18[..., 2]] + [w18[..., 3]] * 5, axis=-1)
    cen = jnp.stack([w18[..., 4]] + [w18[..., 5]] * 5, axis=-1)
    sym = w18[..., 6:18].reshape(w18.shape[:-1] + (2, 6))
    sym = sym[..., jnp.asarray(_ROLL)]
    sym = sym.reshape(w18.shape[:-1] + (10, 6))
    return jnp.concatenate(
        [top[..., None, :], bot[..., None, :], sym, cen[..., None, :]],
        axis=-2)


@jax.jit
def _run(neighbor_index, vertices, W, W_dir):
    v0 = vertices[0]
    ctr = jnp.pad(v0.T, ((0, 0), (0, P_PAD - V)))
    table = ctr[:, :VPAD].reshape(-1)
    nidx = neighbor_index[0].reshape(-1).astype(jnp.int32)
    nidx = jnp.pad(nidx, (0, B_PAD - B_TOT))

    gathered = _get_sc_gather()(table, nidx)
    nbr = gathered.reshape(3, B_PAD)[:, :B_TOT].reshape(3, V, N)
    nbr = jnp.pad(nbr.transpose(0, 2, 1), ((0, 0), (0, 0), (0, P_PAD - V)))

    inv = jnp.asarray(_INV)
    wdc = _rows13(W_dir.sum(0))
    amat = wdc[:, inv].transpose(1, 0, 2)
    a2 = jnp.stack([jnp.stack([wdc[k, 0] - wdc[k, 1], wdc[k, 1]])
                    for k in (0, 1, 12)])
    j0 = jnp.asarray(_CC[:, 0].astype(np.int32))
    wc = _rows13(W.sum(1))
    b2 = wc[:, :, inv].transpose(2, 1, 3, 0)
    b2 = b2.reshape(12, 78, 128)
    b2 = jnp.pad(b2, ((0, 0), (0, 2), (0, 0)))

    out6 = _tc_call(nbr, ctr, amat, a2, j0, b2)
    return out6[:, :, :V].transpose(1, 2, 0)[None]


def kernel(neighbor_index, vertices, W, W_dir):
    return _run(neighbor_index, vertices, W, W_dir)


# speedup vs baseline: 1.2720x; 1.2720x over previous
import functools
import math

import jax
import jax.numpy as jnp
import numpy as np
from jax import lax
from jax.experimental import pallas as pl
from jax.experimental.pallas import tpu as pltpu
from jax.experimental.pallas import tpu_sc as plsc

_phi = (1.0 + np.sqrt(5.0)) / 2.0
_verts = np.array([[-1.0, _phi, 0.0], [1.0, _phi, 0.0], [-1.0, -_phi, 0.0],
                   [1.0, -_phi, 0.0], [0.0, -1.0, _phi], [0.0, 1.0, _phi],
                   [0.0, -1.0, -_phi], [0.0, 1.0, -_phi], [_phi, 0.0, -1.0],
                   [_phi, 0.0, 1.0], [-_phi, 0.0, -1.0], [-_phi, 0.0, 1.0]],
                  dtype=np.float32)
_VS = _verts / np.linalg.norm(_verts, axis=-1, keepdims=True)
_ROLL = np.array([[0, 1, 2, 3, 4, 5], [0, 2, 3, 4, 5, 1], [0, 3, 4, 5, 1, 2],
                  [0, 4, 5, 1, 2, 3], [0, 5, 1, 2, 3, 4]])
_CC = np.array([[0, 1, 2, 3, 4, 5], [0, 5, 4, 3, 2, 1], [2, 5, 4, 1, 0, 3],
                [5, 0, 1, 3, 2, 4], [3, 5, 2, 0, 4, 1], [4, 0, 5, 2, 1, 3],
                [1, 3, 4, 2, 0, 5], [1, 0, 2, 4, 3, 5], [4, 1, 2, 5, 0, 3],
                [3, 1, 4, 0, 2, 5], [5, 1, 0, 4, 2, 3], [2, 4, 5, 3, 0, 1]])
_C2V = np.array([[0, 1], [6, 7], [2, 11], [4, 9], [5, 8], [3, 10]])
_INV = np.argsort(_CC, axis=-1)

V = 10000
N = 16
P_PAD = 10240
BP = 1024
NBLK = P_PAD // BP
B_TOT = V * N
NW = 32
B_PAD = 163840
BPW = B_PAD // NW
VPAD = 10240


def _sc_gather_build():
    mesh = plsc.VectorSubcoreMesh(core_axis_name="c", subcore_axis_name="s")

    @functools.partial(
        pl.kernel,
        mesh=mesh,
        compiler_params=pltpu.CompilerParams(needs_layout_passes=False),
        out_type=jax.ShapeDtypeStruct((3 * B_PAD,), jnp.float32),
        scratch_types=[
            pltpu.VMEM((3 * VPAD,), jnp.float32),
            pltpu.VMEM((BPW,), jnp.int32),
            pltpu.VMEM((3 * BPW,), jnp.float32),
        ],
    )
    def sc_gather(table_hbm, idx_hbm, out_hbm, tab_v, idx_v, rows_v):
        wid = lax.axis_index("s") * 2 + lax.axis_index("c")
        pltpu.sync_copy(table_hbm, tab_v)
        pltpu.sync_copy(idx_hbm.at[pl.ds(wid * BPW, BPW)], idx_v)

        def body(j, carry):
            o = pl.multiple_of(j * 16, 16)
            iv = idx_v[pl.ds(o, 16)]
            for c in range(3):
                g = plsc.load_gather(tab_v, [iv + c * VPAD])
                rows_v[pl.ds(pl.multiple_of(c * BPW + o, 16), 16)] = g
            return carry

        lax.fori_loop(0, BPW // 16, body, 0)
        for c in range(3):
            pltpu.sync_copy(
                rows_v.at[pl.ds(c * BPW, BPW)],
                out_hbm.at[pl.ds(c * B_PAD + wid * BPW, BPW)],
            )

    return sc_gather


_sc_cache = []


def _get_sc_gather():
    if not _sc_cache:
        _sc_cache.append(_sc_gather_build())
    return _sc_cache[0]


def _tc_body(nbr_ref, ctr_ref, a_ref, a2_ref, j0_ref, b2_ref, out_ref,
             nf_ref, de_ref):

    def put_nf(r, kiq, act, d):
        part = act[:8] * d[:8] + act[8:] * d[8:]
        nf_ref[r, pl.ds(kiq * 8, 8), :] = part.astype(jnp.bfloat16)
    nbr = nbr_ref[...]
    ctr = ctr_ref[...]
    nd0 = nbr[0] - ctr[0][None, :]
    nd1 = nbr[1] - ctr[1][None, :]
    nd2 = nbr[2] - ctr[2][None, :]
    n2 = nd0 * nd0 + nd1 * nd1 + nd2 * nd2
    rn = lax.rsqrt(jnp.maximum(n2, 1e-24))
    dv = []
    for v in range(12):
        c0, c1, c2 = float(_VS[v, 0]), float(_VS[v, 1]), float(_VS[v, 2])
        dv.append((c0 * nd0 + c1 * nd1 + c2 * nd2) * rn)
    de6 = [jnp.maximum(dv[int(_C2V[c, 0])], dv[int(_C2V[c, 1])])
           for c in range(6)]
    s_all = ((de6[0] + de6[1]) + (de6[2] + de6[3])) + (de6[4] + de6[5])
    for c in range(6):
        de_ref[c] = de6[c]
    de_ref[6] = s_all

    nf_ref[:, 624:640, :] = jnp.zeros((12, 16, BP), jnp.bfloat16)

    def piv_body(r, carry):
        j0 = j0_ref[r]
        dj = de_ref[j0]
        s = de_ref[6]
        for t, k in ((0, 0), (1, 1), (2, 12)):
            act = jnp.maximum(a2_ref[t, 0] * dj + a2_ref[t, 1] * s, 0.0)
            for i in range(6):
                put_nf(r, k * 6 + i, act, de6[i])
        return carry

    lax.fori_loop(0, 12, piv_body, 0)

    def sym_body(rm, carry):
        r = rm // 5
        k = (rm - r * 5) * 2 + 2
        act1 = (a_ref[r, k, 0] * de6[0] + a_ref[r, k, 1] * de6[1] +
                a_ref[r, k, 2] * de6[2] + a_ref[r, k, 3] * de6[3] +
                a_ref[r, k, 4] * de6[4] + a_ref[r, k, 5] * de6[5])
        act1 = jnp.maximum(act1, 0.0)
        k2 = k + 1
        act2 = (a_ref[r, k2, 0] * de6[0] + a_ref[r, k2, 1] * de6[1] +
                a_ref[r, k2, 2] * de6[2] + a_ref[r, k2, 3] * de6[3] +
                a_ref[r, k2, 4] * de6[4] + a_ref[r, k2, 5] * de6[5])
        act2 = jnp.maximum(act2, 0.0)
        for i in range(6):
            d = de6[i]
            put_nf(r, k * 6 + i, act1, d)
            put_nf(r, k2 * 6 + i, act2, d)
        return carry

    lax.fori_loop(0, 60, sym_body, 0)

    dims = (((0,), (0,)), ((), ()))
    for c in range(6):
        r0, r1 = int(_C2V[c, 0]), int(_C2V[c, 1])
        foa = lax.dot_general(b2_ref[r0], nf_ref[r0], dims,
                              preferred_element_type=jnp.float32)
        fob = lax.dot_general(b2_ref[r1], nf_ref[r1], dims,
                              preferred_element_type=jnp.float32)
        out_ref[c] = jnp.maximum(foa, fob)


def _tc_build(interpret=False):
    return pl.pallas_call(
        _tc_body,
        grid=(NBLK,),
        in_specs=[
            pl.BlockSpec((3, 16, BP), lambda i: (0, 0, i)),
            pl.BlockSpec((3, BP), lambda i: (0, i)),
            pl.BlockSpec(memory_space=pltpu.SMEM),
            pl.BlockSpec(memory_space=pltpu.SMEM),
            pl.BlockSpec(memory_space=pltpu.SMEM),
            pl.BlockSpec((12, 640, 128), lambda i: (0, 0, 0)),
        ],
        out_specs=pl.BlockSpec((6, 128, BP), lambda i: (0, 0, i)),
        out_shape=jax.ShapeDtypeStruct((6, 128, P_PAD), jnp.float32),
        scratch_shapes=[
            pltpu.VMEM((12, 640, BP), jnp.bfloat16),
            pltpu.VMEM((7, 16, BP), jnp.float32),
        ],
        interpret=interpret,
    )


_tc_call = _tc_build()


def _rows13(w18):
    top = jnp.stack([w18[..., 0]] + [w18[..., 1]] * 5, axis=-1)
    bot = jnp.stack([w18[..., 2]] + [w18[..., 3]] * 5, axis=-1)
    cen = jnp.stack([w18[..., 4]] + [w18[..., 5]] * 5, axis=-1)
    sym = w18[..., 6:18].reshape(w18.shape[:-1] + (2, 6))
    sym = sym[..., jnp.asarray(_ROLL)]
    sym = sym.reshape(w18.shape[:-1] + (10, 6))
    return jnp.concatenate(
        [top[..., None, :], bot[..., None, :], sym, cen[..., None, :]],
        axis=-2)


@jax.jit
def _run(neighbor_index, vertices, W, W_dir):
    v0 = vertices[0]
    ctr = jnp.pad(v0.T, ((0, 0), (0, P_PAD - V)))
    table = ctr[:, :VPAD].reshape(-1)
    nidx = neighbor_index[0].reshape(-1).astype(jnp.int32)
    nidx = jnp.pad(nidx, (0, B_PAD - B_TOT))

    gathered = _get_sc_gather()(table, nidx)
    nbr = gathered.reshape(3, B_PAD)[:, :B_TOT].reshape(3, V, N)
    nbr = jnp.pad(nbr.transpose(0, 2, 1), ((0, 0), (0, 0), (0, P_PAD - V)))

    inv = jnp.asarray(_INV)
    wdc = _rows13(W_dir.sum(0))
    amat = wdc[:, inv].transpose(1, 0, 2)
    a2 = jnp.stack([jnp.stack([wdc[k, 0] - wdc[k, 1], wdc[k, 1]])
                    for k in (0, 1, 12)])
    j0 = jnp.asarray(_CC[:, 0].astype(np.int32))
    wc = _rows13(W.sum(1))
    b2 = wc[:, :, inv].transpose(2, 1, 3, 0)
    b2 = b2.reshape(12, 78, 128)
    b2 = jnp.pad(b2, ((0, 0), (0, 2), (0, 0)))
    b2 = jnp.repeat(b2, 8, axis=1).astype(jnp.bfloat16)

    out6 = _tc_call(nbr, ctr, amat, a2, j0, b2)
    return out6[:, :, :V].transpose(1, 2, 0)[None]


def kernel(neighbor_index, vertices, W, W_dir):
    return _run(neighbor_index, vertices, W, W_dir)


# speedup vs baseline: 1.3635x; 1.0720x over previous
import functools
import math

import jax
import jax.numpy as jnp
import numpy as np
from jax import lax
from jax.experimental import pallas as pl
from jax.experimental.pallas import tpu as pltpu
from jax.experimental.pallas import tpu_sc as plsc

_phi = (1.0 + np.sqrt(5.0)) / 2.0
_verts = np.array([[-1.0, _phi, 0.0], [1.0, _phi, 0.0], [-1.0, -_phi, 0.0],
                   [1.0, -_phi, 0.0], [0.0, -1.0, _phi], [0.0, 1.0, _phi],
                   [0.0, -1.0, -_phi], [0.0, 1.0, -_phi], [_phi, 0.0, -1.0],
                   [_phi, 0.0, 1.0], [-_phi, 0.0, -1.0], [-_phi, 0.0, 1.0]],
                  dtype=np.float32)
_VS = _verts / np.linalg.norm(_verts, axis=-1, keepdims=True)
_ROLL = np.array([[0, 1, 2, 3, 4, 5], [0, 2, 3, 4, 5, 1], [0, 3, 4, 5, 1, 2],
                  [0, 4, 5, 1, 2, 3], [0, 5, 1, 2, 3, 4]])
_CC = np.array([[0, 1, 2, 3, 4, 5], [0, 5, 4, 3, 2, 1], [2, 5, 4, 1, 0, 3],
                [5, 0, 1, 3, 2, 4], [3, 5, 2, 0, 4, 1], [4, 0, 5, 2, 1, 3],
                [1, 3, 4, 2, 0, 5], [1, 0, 2, 4, 3, 5], [4, 1, 2, 5, 0, 3],
                [3, 1, 4, 0, 2, 5], [5, 1, 0, 4, 2, 3], [2, 4, 5, 3, 0, 1]])
_C2V = np.array([[0, 1], [6, 7], [2, 11], [4, 9], [5, 8], [3, 10]])
_INV = np.argsort(_CC, axis=-1)

V = 10000
N = 16
P_PAD = 10240
BP = 1024
NBLK = P_PAD // BP
B_TOT = V * N
NW = 32
B_PAD = 163840
BPW = B_PAD // NW
VPAD = 10240


def _sc_gather_build():
    mesh = plsc.VectorSubcoreMesh(core_axis_name="c", subcore_axis_name="s")

    @functools.partial(
        pl.kernel,
        mesh=mesh,
        compiler_params=pltpu.CompilerParams(needs_layout_passes=False),
        out_type=jax.ShapeDtypeStruct((3 * B_PAD,), jnp.float32),
        scratch_types=[
            pltpu.VMEM((3 * VPAD,), jnp.float32),
            pltpu.VMEM((BPW,), jnp.int32),
            pltpu.VMEM((3 * BPW,), jnp.float32),
        ],
    )
    def sc_gather(table_hbm, idx_hbm, out_hbm, tab_v, idx_v, rows_v):
        wid = lax.axis_index("s") * 2 + lax.axis_index("c")
        pltpu.sync_copy(table_hbm, tab_v)
        pltpu.sync_copy(idx_hbm.at[pl.ds(wid * BPW, BPW)], idx_v)

        def body(j, carry):
            o = pl.multiple_of(j * 16, 16)
            iv = idx_v[pl.ds(o, 16)]
            for c in range(3):
                g = plsc.load_gather(tab_v, [iv + c * VPAD])
                rows_v[pl.ds(pl.multiple_of(c * BPW + o, 16), 16)] = g
            return carry

        lax.fori_loop(0, BPW // 16, body, 0)
        for c in range(3):
            pltpu.sync_copy(
                rows_v.at[pl.ds(c * BPW, BPW)],
                out_hbm.at[pl.ds(c * B_PAD + wid * BPW, BPW)],
            )

    return sc_gather


_sc_cache = []


def _get_sc_gather():
    if not _sc_cache:
        _sc_cache.append(_sc_gather_build())
    return _sc_cache[0]


def _tc_body(nbr_ref, ctr_ref, a_ref, a2_ref, j0_ref, b2_ref, out_ref,
             nf_ref, de_ref):

    def put_nf(r, kiq, act, d):
        part = act[:8] * d[:8] + act[8:] * d[8:]
        nf_ref[r, pl.ds(kiq * 8, 8), :] = part.astype(jnp.bfloat16)
    nbr = nbr_ref[...]
    ctr = ctr_ref[...]
    nd0 = nbr[0] - ctr[0][None, :]
    nd1 = nbr[1] - ctr[1][None, :]
    nd2 = nbr[2] - ctr[2][None, :]
    n2 = nd0 * nd0 + nd1 * nd1 + nd2 * nd2
    rn = lax.rsqrt(jnp.maximum(n2, 1e-24))
    dv = []
    for v in range(12):
        c0, c1, c2 = float(_VS[v, 0]), float(_VS[v, 1]), float(_VS[v, 2])
        dv.append((c0 * nd0 + c1 * nd1 + c2 * nd2) * rn)
    de6 = [jnp.maximum(dv[int(_C2V[c, 0])], dv[int(_C2V[c, 1])])
           for c in range(6)]
    s_all = ((de6[0] + de6[1]) + (de6[2] + de6[3])) + (de6[4] + de6[5])
    for c in range(6):
        de_ref[c] = de6[c]
    de_ref[6] = s_all

    nf_ref[:, 624:640, :] = jnp.zeros((12, 16, BP), jnp.bfloat16)

    def piv_body(r, carry):
        j0 = j0_ref[r]
        dj = de_ref[j0]
        s = de_ref[6]
        for t, k in ((0, 0), (1, 1), (2, 12)):
            act = jnp.maximum(a2_ref[t, 0] * dj + a2_ref[t, 1] * s, 0.0)
            for i in range(6):
                put_nf(r, k * 6 + i, act, de6[i])
        return carry

    lax.fori_loop(0, 12, piv_body, 0)

    def sym_body(r, carry):
        for k in range(2, 12):
            act = (a_ref[r, k, 0] * de6[0] + a_ref[r, k, 1] * de6[1] +
                   a_ref[r, k, 2] * de6[2] + a_ref[r, k, 3] * de6[3] +
                   a_ref[r, k, 4] * de6[4] + a_ref[r, k, 5] * de6[5])
            act = jnp.maximum(act, 0.0)
            for i in range(6):
                put_nf(r, k * 6 + i, act, de6[i])
        return carry

    lax.fori_loop(0, 12, sym_body, 0)

    dims = (((0,), (0,)), ((), ()))
    for c in range(6):
        r0, r1 = int(_C2V[c, 0]), int(_C2V[c, 1])
        foa = lax.dot_general(b2_ref[r0], nf_ref[r0], dims,
                              preferred_element_type=jnp.float32)
        fob = lax.dot_general(b2_ref[r1], nf_ref[r1], dims,
                              preferred_element_type=jnp.float32)
        out_ref[c] = jnp.maximum(foa, fob)


def _tc_build(interpret=False):
    return pl.pallas_call(
        _tc_body,
        grid=(NBLK,),
        in_specs=[
            pl.BlockSpec((3, 16, BP), lambda i: (0, 0, i)),
            pl.BlockSpec((3, BP), lambda i: (0, i)),
            pl.BlockSpec(memory_space=pltpu.SMEM),
            pl.BlockSpec(memory_space=pltpu.SMEM),
            pl.BlockSpec(memory_space=pltpu.SMEM),
            pl.BlockSpec((12, 640, 128), lambda i: (0, 0, 0)),
        ],
        out_specs=pl.BlockSpec((6, 128, BP), lambda i: (0, 0, i)),
        out_shape=jax.ShapeDtypeStruct((6, 128, P_PAD), jnp.float32),
        scratch_shapes=[
            pltpu.VMEM((12, 640, BP), jnp.bfloat16),
            pltpu.VMEM((7, 16, BP), jnp.float32),
        ],
        interpret=interpret,
    )


_tc_call = _tc_build()


def _rows13(w18):
    top = jnp.stack([w18[..., 0]] + [w18[..., 1]] * 5, axis=-1)
    bot = jnp.stack([w18[..., 2]] + [w18[..., 3]] * 5, axis=-1)
    cen = jnp.stack([w18[..., 4]] + [w18[..., 5]] * 5, axis=-1)
    sym = w18[..., 6:18].reshape(w18.shape[:-1] + (2, 6))
    sym = sym[..., jnp.asarray(_ROLL)]
    sym = sym.reshape(w18.shape[:-1] + (10, 6))
    return jnp.concatenate(
        [top[..., None, :], bot[..., None, :], sym, cen[..., None, :]],
        axis=-2)


@jax.jit
def _run(neighbor_index, vertices, W, W_dir):
    v0 = vertices[0]
    ctr = jnp.pad(v0.T, ((0, 0), (0, P_PAD - V)))
    table = ctr[:, :VPAD].reshape(-1)
    nidx = neighbor_index[0].reshape(-1).astype(jnp.int32)
    nidx = jnp.pad(nidx, (0, B_PAD - B_TOT))

    gathered = _get_sc_gather()(table, nidx)
    nbr = gathered.reshape(3, B_PAD)[:, :B_TOT].reshape(3, V, N)
    nbr = jnp.pad(nbr.transpose(0, 2, 1), ((0, 0), (0, 0), (0, P_PAD - V)))

    inv = jnp.asarray(_INV)
    wdc = _rows13(W_dir.sum(0))
    amat = wdc[:, inv].transpose(1, 0, 2)
    a2 = jnp.stack([jnp.stack([wdc[k, 0] - wdc[k, 1], wdc[k, 1]])
                    for k in (0, 1, 12)])
    j0 = jnp.asarray(_CC[:, 0].astype(np.int32))
    wc = _rows13(W.sum(1))
    b2 = wc[:, :, inv].transpose(2, 1, 3, 0)
    b2 = b2.reshape(12, 78, 128)
    b2 = jnp.pad(b2, ((0, 0), (0, 2), (0, 0)))
    b2 = jnp.repeat(b2, 8, axis=1).astype(jnp.bfloat16)

    out6 = _tc_call(nbr, ctr, amat, a2, j0, b2)
    return out6[:, :, :V].transpose(1, 2, 0)[None]


def kernel(neighbor_index, vertices, W, W_dir):
    return _run(neighbor_index, vertices, W, W_dir)


# speedup vs baseline: 1.3709x; 1.0054x over previous
import functools
import math

import jax
import jax.numpy as jnp
import numpy as np
from jax import lax
from jax.experimental import pallas as pl
from jax.experimental.pallas import tpu as pltpu
from jax.experimental.pallas import tpu_sc as plsc

_phi = (1.0 + np.sqrt(5.0)) / 2.0
_verts = np.array([[-1.0, _phi, 0.0], [1.0, _phi, 0.0], [-1.0, -_phi, 0.0],
                   [1.0, -_phi, 0.0], [0.0, -1.0, _phi], [0.0, 1.0, _phi],
                   [0.0, -1.0, -_phi], [0.0, 1.0, -_phi], [_phi, 0.0, -1.0],
                   [_phi, 0.0, 1.0], [-_phi, 0.0, -1.0], [-_phi, 0.0, 1.0]],
                  dtype=np.float32)
_VS = _verts / np.linalg.norm(_verts, axis=-1, keepdims=True)
_ROLL = np.array([[0, 1, 2, 3, 4, 5], [0, 2, 3, 4, 5, 1], [0, 3, 4, 5, 1, 2],
                  [0, 4, 5, 1, 2, 3], [0, 5, 1, 2, 3, 4]])
_CC = np.array([[0, 1, 2, 3, 4, 5], [0, 5, 4, 3, 2, 1], [2, 5, 4, 1, 0, 3],
                [5, 0, 1, 3, 2, 4], [3, 5, 2, 0, 4, 1], [4, 0, 5, 2, 1, 3],
                [1, 3, 4, 2, 0, 5], [1, 0, 2, 4, 3, 5], [4, 1, 2, 5, 0, 3],
                [3, 1, 4, 0, 2, 5], [5, 1, 0, 4, 2, 3], [2, 4, 5, 3, 0, 1]])
_C2V = np.array([[0, 1], [6, 7], [2, 11], [4, 9], [5, 8], [3, 10]])
_INV = np.argsort(_CC, axis=-1)

V = 10000
N = 16
P_PAD = 10240
BP = 1024
NBLK = P_PAD // BP
B_TOT = V * N
NW = 32
B_PAD = 163840
BPW = B_PAD // NW
VPAD = 10240


def _sc_gather_build():
    mesh = plsc.VectorSubcoreMesh(core_axis_name="c", subcore_axis_name="s")

    @functools.partial(
        pl.kernel,
        mesh=mesh,
        compiler_params=pltpu.CompilerParams(needs_layout_passes=False),
        out_type=jax.ShapeDtypeStruct((3 * B_PAD,), jnp.float32),
        scratch_types=[
            pltpu.VMEM((3 * VPAD,), jnp.float32),
            pltpu.VMEM((BPW,), jnp.int32),
            pltpu.VMEM((3 * BPW,), jnp.float32),
        ],
    )
    def sc_gather(table_hbm, idx_hbm, out_hbm, tab_v, idx_v, rows_v):
        wid = lax.axis_index("s") * 2 + lax.axis_index("c")
        pltpu.sync_copy(table_hbm, tab_v)
        pltpu.sync_copy(idx_hbm.at[pl.ds(wid * BPW, BPW)], idx_v)

        def body(j, carry):
            o = pl.multiple_of(j * 16, 16)
            iv = idx_v[pl.ds(o, 16)]
            for c in range(3):
                g = plsc.load_gather(tab_v, [iv + c * VPAD])
                rows_v[pl.ds(pl.multiple_of(c * BPW + o, 16), 16)] = g
            return carry

        lax.fori_loop(0, BPW // 16, body, 0)
        for c in range(3):
            pltpu.sync_copy(
                rows_v.at[pl.ds(c * BPW, BPW)],
                out_hbm.at[pl.ds(c * B_PAD + wid * BPW, BPW)],
            )

    return sc_gather


_sc_cache = []


def _get_sc_gather():
    if not _sc_cache:
        _sc_cache.append(_sc_gather_build())
    return _sc_cache[0]


def _tc_body(nbr_ref, ctr_ref, a_ref, a2_ref, j0_ref, b2_ref, out_ref,
             nf_ref, de_ref):

    def put_nf(r, kiq, act, d):
        part = act[:8] * d[:8] + act[8:] * d[8:]
        nf_ref[r, pl.ds(kiq * 8, 8), :] = part.astype(jnp.bfloat16)
    nbr = nbr_ref[...]
    ctr = ctr_ref[...]
    nd0 = nbr[0] - ctr[0][None, :]
    nd1 = nbr[1] - ctr[1][None, :]
    nd2 = nbr[2] - ctr[2][None, :]
    n2 = nd0 * nd0 + nd1 * nd1 + nd2 * nd2
    rn = lax.rsqrt(jnp.maximum(n2, 1e-24))
    dv = []
    for v in range(12):
        c0, c1, c2 = float(_VS[v, 0]), float(_VS[v, 1]), float(_VS[v, 2])
        dv.append((c0 * nd0 + c1 * nd1 + c2 * nd2) * rn)
    de6 = [jnp.maximum(dv[int(_C2V[c, 0])], dv[int(_C2V[c, 1])])
           for c in range(6)]
    s_all = ((de6[0] + de6[1]) + (de6[2] + de6[3])) + (de6[4] + de6[5])
    for c in range(6):
        de_ref[c] = de6[c]
    de_ref[6] = s_all

    def row_body(r, carry):
        j0 = j0_ref[r]
        dj = de_ref[j0]
        s = de_ref[6]
        for t, k in ((0, 0), (1, 1), (2, 12)):
            act = jnp.maximum(a2_ref[t, 0] * dj + a2_ref[t, 1] * s, 0.0)
            for i in range(6):
                put_nf(r, k * 6 + i, act, de6[i])
        for k in range(2, 12):
            act = (a_ref[r, k, 0] * de6[0] + a_ref[r, k, 1] * de6[1] +
                   a_ref[r, k, 2] * de6[2] + a_ref[r, k, 3] * de6[3] +
                   a_ref[r, k, 4] * de6[4] + a_ref[r, k, 5] * de6[5])
            act = jnp.maximum(act, 0.0)
            for i in range(6):
                put_nf(r, k * 6 + i, act, de6[i])
        return carry

    lax.fori_loop(0, 12, row_body, 0)

    dims = (((0,), (0,)), ((), ()))
    for c in range(6):
        r0, r1 = int(_C2V[c, 0]), int(_C2V[c, 1])
        foa = lax.dot_general(b2_ref[r0], nf_ref[r0], dims,
                              preferred_element_type=jnp.float32)
        fob = lax.dot_general(b2_ref[r1], nf_ref[r1], dims,
                              preferred_element_type=jnp.float32)
        out_ref[c] = jnp.maximum(foa, fob)


def _tc_build(interpret=False):
    return pl.pallas_call(
        _tc_body,
        grid=(NBLK,),
        in_specs=[
            pl.BlockSpec((3, 16, BP), lambda i: (0, 0, i)),
            pl.BlockSpec((3, BP), lambda i: (0, i)),
            pl.BlockSpec(memory_space=pltpu.SMEM),
            pl.BlockSpec(memory_space=pltpu.SMEM),
            pl.BlockSpec(memory_space=pltpu.SMEM),
            pl.BlockSpec((12, 624, 128), lambda i: (0, 0, 0)),
        ],
        out_specs=pl.BlockSpec((6, 128, BP), lambda i: (0, 0, i)),
        out_shape=jax.ShapeDtypeStruct((6, 128, P_PAD), jnp.float32),
        scratch_shapes=[
            pltpu.VMEM((12, 624, BP), jnp.bfloat16),
            pltpu.VMEM((7, 16, BP), jnp.float32),
        ],
        interpret=interpret,
    )


_tc_call = _tc_build()


def _rows13(w18):
    top = jnp.stack([w18[..., 0]] + [w18[..., 1]] * 5, axis=-1)
    bot = jnp.stack([w18[..., 2]] + [w18[..., 3]] * 5, axis=-1)
    cen = jnp.stack([w18[..., 4]] + [w18[..., 5]] * 5, axis=-1)
    sym = w18[..., 6:18].reshape(w18.shape[:-1] + (2, 6))
    sym = sym[..., jnp.asarray(_ROLL)]
    sym = sym.reshape(w18.shape[:-1] + (10, 6))
    return jnp.concatenate(
        [top[..., None, :], bot[..., None, :], sym, cen[..., None, :]],
        axis=-2)


@jax.jit
def _run(neighbor_index, vertices, W, W_dir):
    v0 = vertices[0]
    ctr = jnp.pad(v0.T, ((0, 0), (0, P_PAD - V)))
    table = ctr[:, :VPAD].reshape(-1)
    nidx = neighbor_index[0].reshape(-1).astype(jnp.int32)
    nidx = jnp.pad(nidx, (0, B_PAD - B_TOT))

    gathered = _get_sc_gather()(table, nidx)
    nbr = gathered.reshape(3, B_PAD)[:, :B_TOT].reshape(3, V, N)
    nbr = jnp.pad(nbr.transpose(0, 2, 1), ((0, 0), (0, 0), (0, P_PAD - V)))

    inv = jnp.asarray(_INV)
    wdc = _rows13(W_dir.sum(0))
    amat = wdc[:, inv].transpose(1, 0, 2)
    a2 = jnp.stack([jnp.stack([wdc[k, 0] - wdc[k, 1], wdc[k, 1]])
                    for k in (0, 1, 12)])
    j0 = jnp.asarray(_CC[:, 0].astype(np.int32))
    wc = _rows13(W.sum(1))
    b2 = wc[:, :, inv].transpose(2, 1, 3, 0)
    b2 = b2.reshape(12, 78, 128)
    b2 = jnp.repeat(b2, 8, axis=1).astype(jnp.bfloat16)

    out6 = _tc_call(nbr, ctr, amat, a2, j0, b2)
    return out6[:, :, :V].transpose(1, 2, 0)[None]


def kernel(neighbor_index, vertices, W, W_dir):
    return _run(neighbor_index, vertices, W, W_dir)


# speedup vs baseline: 1.5693x; 1.1447x over previous
import functools
import math

import jax
import jax.numpy as jnp
import numpy as np
from jax import lax
from jax.experimental import pallas as pl
from jax.experimental.pallas import tpu as pltpu
from jax.experimental.pallas import tpu_sc as plsc

_phi = (1.0 + np.sqrt(5.0)) / 2.0
_verts = np.array([[-1.0, _phi, 0.0], [1.0, _phi, 0.0], [-1.0, -_phi, 0.0],
                   [1.0, -_phi, 0.0], [0.0, -1.0, _phi], [0.0, 1.0, _phi],
                   [0.0, -1.0, -_phi], [0.0, 1.0, -_phi], [_phi, 0.0, -1.0],
                   [_phi, 0.0, 1.0], [-_phi, 0.0, -1.0], [-_phi, 0.0, 1.0]],
                  dtype=np.float32)
_VS = _verts / np.linalg.norm(_verts, axis=-1, keepdims=True)
_ROLL = np.array([[0, 1, 2, 3, 4, 5], [0, 2, 3, 4, 5, 1], [0, 3, 4, 5, 1, 2],
                  [0, 4, 5, 1, 2, 3], [0, 5, 1, 2, 3, 4]])
_CC = np.array([[0, 1, 2, 3, 4, 5], [0, 5, 4, 3, 2, 1], [2, 5, 4, 1, 0, 3],
                [5, 0, 1, 3, 2, 4], [3, 5, 2, 0, 4, 1], [4, 0, 5, 2, 1, 3],
                [1, 3, 4, 2, 0, 5], [1, 0, 2, 4, 3, 5], [4, 1, 2, 5, 0, 3],
                [3, 1, 4, 0, 2, 5], [5, 1, 0, 4, 2, 3], [2, 4, 5, 3, 0, 1]])
_C2V = np.array([[0, 1], [6, 7], [2, 11], [4, 9], [5, 8], [3, 10]])
_INV = np.argsort(_CC, axis=-1)

V = 10000
N = 16
P_PAD = 10240
BP = 1024
NBLK = P_PAD // BP
B_TOT = V * N
NW = 32
B_PAD = 163840
BPW = B_PAD // NW
VPAD = 10240


def _sc_gather_build():
    mesh = plsc.VectorSubcoreMesh(core_axis_name="c", subcore_axis_name="s")

    @functools.partial(
        pl.kernel,
        mesh=mesh,
        compiler_params=pltpu.CompilerParams(needs_layout_passes=False),
        out_type=jax.ShapeDtypeStruct((3 * B_PAD,), jnp.float32),
        scratch_types=[
            pltpu.VMEM((3 * VPAD,), jnp.float32),
            pltpu.VMEM((BPW,), jnp.int32),
            pltpu.VMEM((3 * BPW,), jnp.float32),
        ],
    )
    def sc_gather(table_hbm, idx_hbm, out_hbm, tab_v, idx_v, rows_v):
        wid = lax.axis_index("s") * 2 + lax.axis_index("c")
        pltpu.sync_copy(table_hbm, tab_v)
        pltpu.sync_copy(idx_hbm.at[pl.ds(wid * BPW, BPW)], idx_v)

        def body(j, carry):
            o = pl.multiple_of(j * 16, 16)
            iv = idx_v[pl.ds(o, 16)]
            for c in range(3):
                g = plsc.load_gather(tab_v, [iv + c * VPAD])
                rows_v[pl.ds(pl.multiple_of(c * BPW + o, 16), 16)] = g
            return carry

        lax.fori_loop(0, BPW // 16, body, 0)
        for c in range(3):
            pltpu.sync_copy(
                rows_v.at[pl.ds(c * BPW, BPW)],
                out_hbm.at[pl.ds(c * B_PAD + wid * BPW, BPW)],
            )

    return sc_gather


_sc_cache = []


def _get_sc_gather():
    if not _sc_cache:
        _sc_cache.append(_sc_gather_build())
    return _sc_cache[0]


def _tc_body(nbr_ref, ctr_ref, a_ref, a2_ref, j0_ref, b2_ref, out_ref,
             nf_ref, de_ref):

    def put_nf(r, kiq, act, d):
        part = act[:8] * d[:8] + act[8:] * d[8:]
        nf_ref[r, pl.ds(kiq * 8, 8), :] = part.astype(jnp.bfloat16)
    nbr = nbr_ref[...]
    ctr = ctr_ref[...]
    nd0 = nbr[0] - ctr[0][None, :]
    nd1 = nbr[1] - ctr[1][None, :]
    nd2 = nbr[2] - ctr[2][None, :]
    n2 = nd0 * nd0 + nd1 * nd1 + nd2 * nd2
    rn = lax.rsqrt(jnp.maximum(n2, 1e-24))
    dv = []
    for v in range(12):
        c0, c1, c2 = float(_VS[v, 0]), float(_VS[v, 1]), float(_VS[v, 2])
        dv.append((c0 * nd0 + c1 * nd1 + c2 * nd2) * rn)
    de6 = [jnp.maximum(dv[int(_C2V[c, 0])], dv[int(_C2V[c, 1])])
           for c in range(6)]
    s_all = ((de6[0] + de6[1]) + (de6[2] + de6[3])) + (de6[4] + de6[5])
    for c in range(6):
        de_ref[c] = de6[c]
    de_ref[6] = s_all

    def row_body(r, carry):
        j0 = j0_ref[r]
        dj = de_ref[j0]
        s = de_ref[6]
        for t, k in ((0, 0), (1, 1), (2, 12)):
            act = jnp.maximum(a2_ref[t, 0] * dj + a2_ref[t, 1] * s, 0.0)
            for i in range(6):
                put_nf(r, k * 6 + i, act, de6[i])
        for k in range(2, 12):
            act = (a_ref[r, k, 0] * de6[0] + a_ref[r, k, 1] * de6[1] +
                   a_ref[r, k, 2] * de6[2] + a_ref[r, k, 3] * de6[3] +
                   a_ref[r, k, 4] * de6[4] + a_ref[r, k, 5] * de6[5])
            act = jnp.maximum(act, 0.0)
            for i in range(6):
                put_nf(r, k * 6 + i, act, de6[i])
        return carry

    lax.fori_loop(0, 12, row_body, 0)

    dims = (((0,), (0,)), ((), ()))
    for c in range(6):
        r0, r1 = int(_C2V[c, 0]), int(_C2V[c, 1])
        foa = lax.dot_general(b2_ref[r0], nf_ref[r0], dims,
                              preferred_element_type=jnp.float32)
        fob = lax.dot_general(b2_ref[r1], nf_ref[r1], dims,
                              preferred_element_type=jnp.float32)
        out_ref[c] = jnp.maximum(foa, fob)


def _tc_build(interpret=False):
    return pl.pallas_call(
        _tc_body,
        grid=(NBLK,),
        in_specs=[
            pl.BlockSpec((3, 16, BP), lambda i: (0, 0, i)),
            pl.BlockSpec((3, BP), lambda i: (0, i)),
            pl.BlockSpec(memory_space=pltpu.SMEM),
            pl.BlockSpec(memory_space=pltpu.SMEM),
            pl.BlockSpec(memory_space=pltpu.SMEM),
            pl.BlockSpec((12, 624, 128), lambda i: (0, 0, 0)),
        ],
        out_specs=pl.BlockSpec((6, 128, BP), lambda i: (0, 0, i)),
        out_shape=jax.ShapeDtypeStruct((6, 128, P_PAD), jnp.float32),
        scratch_shapes=[
            pltpu.VMEM((12, 624, BP), jnp.bfloat16),
            pltpu.VMEM((7, 16, BP), jnp.float32),
        ],
        interpret=interpret,
    )


_tc_call = _tc_build()


def _rows13(w18):
    top = jnp.stack([w18[..., 0]] + [w18[..., 1]] * 5, axis=-1)
    bot = jnp.stack([w18[..., 2]] + [w18[..., 3]] * 5, axis=-1)
    cen = jnp.stack([w18[..., 4]] + [w18[..., 5]] * 5, axis=-1)
    sym = w18[..., 6:18].reshape(w18.shape[:-1] + (2, 6))
    sym = sym[..., jnp.asarray(_ROLL)]
    sym = sym.reshape(w18.shape[:-1] + (10, 6))
    return jnp.concatenate(
        [top[..., None, :], bot[..., None, :], sym, cen[..., None, :]],
        axis=-2)


@jax.jit
def _run(neighbor_index, vertices, W, W_dir):
    v0 = vertices[0]
    ctr = jnp.pad(v0.T, ((0, 0), (0, P_PAD - V)))
    table = ctr[:, :VPAD].reshape(-1)
    nidx = neighbor_index[0].T.astype(jnp.int32)
    nidx = jnp.pad(nidx, ((0, 0), (0, P_PAD - V))).reshape(-1)

    gathered = _get_sc_gather()(table, nidx)
    nbr = gathered.reshape(3, N, P_PAD)

    inv = jnp.asarray(_INV)
    wdc = _rows13(W_dir.sum(0))
    amat = wdc[:, inv].transpose(1, 0, 2)
    a2 = jnp.stack([jnp.stack([wdc[k, 0] - wdc[k, 1], wdc[k, 1]])
                    for k in (0, 1, 12)])
    j0 = jnp.asarray(_CC[:, 0].astype(np.int32))
    wc = _rows13(W.sum(1))
    b2 = wc[:, :, inv].transpose(2, 1, 3, 0)
    b2 = b2.reshape(12, 78, 128)
    b2 = jnp.repeat(b2, 8, axis=1).astype(jnp.bfloat16)

    out6 = _tc_call(nbr, ctr, amat, a2, j0, b2)
    return out6[:, :, :V].transpose(1, 2, 0)[None]


def kernel(neighbor_index, vertices, W, W_dir):
    return _run(neighbor_index, vertices, W, W_dir)
